# baseline (speedup 1.0000x reference)
import functools

import jax
import jax.numpy as jnp
from jax import lax
from jax.experimental import pallas as pl
from jax.experimental.pallas import tpu as pltpu
from jax.experimental.pallas import tpu_sc as plsc

VOCAB_SIZE = 1000000
D_MODEL = 64
NUM_CORES = 2
NUM_SUBCORES = 16
NUM_WORKERS = NUM_CORES * NUM_SUBCORES

SLAB = 128
NSLAB_FULL = VOCAB_SIZE // SLAB
SLAB_REM = VOCAB_SIZE - NSLAB_FULL * SLAB
REM_V0 = NSLAB_FULL * SLAB

N_BATCH = 16384
N_SEQ = 50
B_TOTAL = N_BATCH * N_SEQ
B_PER_W = B_TOTAL // NUM_WORKERS
BATCH_PER_W = N_BATCH // NUM_WORKERS
B_BLK = 128
R_BLK = 1
CHUNK = B_BLK * R_BLK
NCHUNK = (BATCH_PER_W // B_BLK) * (N_SEQ // R_BLK)


def _worker_id():
    return lax.axis_index("s") * NUM_CORES + lax.axis_index("c")


@jax.jit
def _prep_kernel(table_t, table_rem):
    mesh = plsc.VectorSubcoreMesh(core_axis_name="c", subcore_axis_name="s")

    @functools.partial(
        pl.kernel,
        mesh=mesh,
        out_type=jax.ShapeDtypeStruct((VOCAB_SIZE, 2 * D_MODEL), jnp.float32),
        compiler_params=pltpu.CompilerParams(use_tc_tiling_on_sc=True, needs_layout_passes=False),
        scratch_types=[
            pltpu.VMEM((D_MODEL, SLAB), jnp.float32),
            pltpu.VMEM((SLAB, 2 * D_MODEL), jnp.float32),
        ],
    )
    def k(tt_hbm, rem_hbm, tdup_hbm, slab_v, tslab_v):
        wid = _worker_id()
        lanes = lax.broadcasted_iota(jnp.int32, (16,), 0)
        n_slabs = jnp.where(wid < NSLAB_FULL % NUM_WORKERS,
                            NSLAB_FULL // NUM_WORKERS + 1,
                            NSLAB_FULL // NUM_WORKERS)

        def transpose_slab(width):
            def do_col(vloc, c2):
                vsplat = jnp.broadcast_to(vloc, (16,))
                for dg in range(D_MODEL // 16):
                    vals = plsc.load_gather(slab_v, [dg * 16 + lanes, vsplat])
                    tslab_v[vloc, pl.ds(dg * 16, 16)] = vals
                return c2

            lax.fori_loop(0, width, do_col, 0, unroll=2)

        def do_slab(s, carry):
            v0 = (wid + s * NUM_WORKERS) * SLAB
            pltpu.sync_copy(tt_hbm.at[:, pl.ds(v0, SLAB)], slab_v)
            transpose_slab(SLAB)
            pltpu.sync_copy(tslab_v, tdup_hbm.at[pl.ds(v0, SLAB)])
            return carry

        lax.fori_loop(0, n_slabs, do_slab, 0)

        @pl.when(wid == NUM_WORKERS - 1)
        def _():
            pltpu.sync_copy(rem_hbm, slab_v)
            transpose_slab(SLAB_REM)
            pltpu.sync_copy(
                tslab_v.at[pl.ds(0, SLAB_REM)],
                tdup_hbm.at[pl.ds(REM_V0, SLAB_REM)],
            )

    return k(table_t, table_rem)


@jax.jit
def _gather_kernel(idx, tdup):
    mesh = plsc.VectorSubcoreMesh(core_axis_name="c", subcore_axis_name="s")

    @functools.partial(
        pl.kernel,
        mesh=mesh,
        out_type=jax.ShapeDtypeStruct((N_SEQ * D_MODEL, N_BATCH), jnp.float32),
        compiler_params=pltpu.CompilerParams(use_tc_tiling_on_sc=True, needs_layout_passes=False),
        scratch_types=[
            pltpu.VMEM((B_PER_W,), jnp.int32),
            pltpu.VMEM((2, CHUNK), jnp.int32),
            pltpu.VMEM((2, CHUNK, 2 * D_MODEL), jnp.float32),
            pltpu.VMEM((R_BLK * D_MODEL, B_BLK), jnp.float32),
            pltpu.SemaphoreType.DMA,
            pltpu.SemaphoreType.DMA,
            pltpu.SemaphoreType.DMA,
        ],
    )
    def k(idx_hbm, tdup_hbm, out_hbm, idxf, idxc, rows_v, tbuf_v, g0, g1, wsem):
        wid = _worker_id()
        base = wid * B_PER_W
        b0w = wid * BATCH_PER_W
        gsem = (g0, g1)
        lanes = lax.broadcasted_iota(jnp.int32, (16,), 0)
        lanes50 = lanes * N_SEQ

        pltpu.sync_copy(idx_hbm.at[pl.ds(base, B_PER_W)], idxf)

        def chunk_coords(c):
            blk = c // (N_SEQ // R_BLK)
            r0 = (c % (N_SEQ // R_BLK)) * R_BLK
            return blk, r0

        def build_idxc(c, buf):
            blk, r0 = chunk_coords(c)

            def do_g(g, c2):
                src0 = (blk * B_BLK + g * 16) * N_SEQ + r0
                vals = plsc.load_gather(idxf, [lanes50 + src0])
                idxc[buf, pl.ds(g * 16, 16)] = vals
                return c2

            lax.fori_loop(0, B_BLK // 16, do_g, 0, unroll=2)

        def gather_desc(b):
            return pltpu.make_async_copy(
                tdup_hbm.at[idxc.at[b]], rows_v.at[b], gsem[b]
            )

        def out_desc(c):
            blk, r0 = chunk_coords(c)
            return pltpu.make_async_copy(
                tbuf_v,
                out_hbm.at[
                    pl.ds(r0 * D_MODEL, R_BLK * D_MODEL),
                    pl.ds(b0w + blk * B_BLK, B_BLK),
                ],
                wsem,
            )

        build_idxc(0, 0)
        gather_desc(0).start()

        def do_group(grp, carry):
            for b in range(2):
                c = grp * 2 + b
                nb = 1 - b

                @pl.when(c + 1 < NCHUNK)
                def _():
                    build_idxc(c + 1, nb)
                    gather_desc(nb).start()

                gather_desc(b).wait()

                @pl.when(c >= 1)
                def _():
                    out_desc(c - 1).wait()

                def do_p(d, c2):
                    dsplat = jnp.broadcast_to(d, (16,))
                    for g in range(B_BLK // 16):
                        vals = plsc.load_gather(
                            rows_v.at[b],
                            [g * 16 + lanes, dsplat],
                        )
                        tbuf_v[d, pl.ds(g * 16, 16)] = vals
                    return c2

                lax.fori_loop(0, R_BLK * D_MODEL, do_p, 0)

                out_desc(c).start()
            return carry

        lax.fori_loop(0, NCHUNK // 2, do_group, 0)
        out_desc(NCHUNK - 1).wait()

    return k(idx, tdup)


def kernel(x, table):
    idx = x.reshape(-1).astype(jnp.int32)
    table_t = table.T
    rem = jnp.pad(table_t[:, REM_V0:], ((0, 0), (0, SLAB - SLAB_REM)))
    tdup = _prep_kernel(table_t, rem)
    out3 = _gather_kernel(idx, tdup)
    return out3.reshape(N_SEQ, D_MODEL, N_BATCH).transpose(2, 0, 1)
